# initial kernel scaffold (unmeasured)
import jax
import jax.numpy as jnp
from jax import lax
from jax.experimental import pallas as pl
from jax.experimental.pallas import tpu as pltpu

N_DEV = 8
B = 64
D = 2048
H_SHARD = 4096
BH = 512


def _layer_body(x_ref, win_ref, wout_ref, o_ref):
    j = pl.program_id(0)
    h = jnp.maximum(
        jnp.dot(x_ref[...], win_ref[...], preferred_element_type=jnp.float32),
        0.0,
    )
    contrib = jnp.dot(h, wout_ref[...], preferred_element_type=jnp.float32)

    @pl.when(j == 0)
    def _():
        o_ref[...] = contrib

    @pl.when(j > 0)
    def _():
        o_ref[...] += contrib


def _mlp_layer(x, win, wout):
    return pl.pallas_call(
        _layer_body,
        grid=(H_SHARD // BH,),
        in_specs=[
            pl.BlockSpec((B, D), lambda j: (0, 0)),
            pl.BlockSpec((D, BH), lambda j: (0, j)),
            pl.BlockSpec((BH, D), lambda j: (j, 0)),
        ],
        out_specs=pl.BlockSpec((B, D), lambda j: (0, 0)),
        out_shape=jax.ShapeDtypeStruct((B, D), jnp.float32),
    )(x, win, wout)


def _ar_body(p_ref, o_ref, comm_ref, send_sems, recv_sems):
    my = lax.axis_index("i")

    barrier_sem = pltpu.get_barrier_semaphore()
    for peer in range(N_DEV):
        @pl.when(my != peer)
        def _():
            pl.semaphore_signal(
                barrier_sem, inc=1,
                device_id=(peer,), device_id_type=pl.DeviceIdType.MESH,
            )
    pl.semaphore_wait(barrier_sem, N_DEV - 1)

    for peer in range(N_DEV):
        @pl.when(my != peer)
        def _():
            rdma = pltpu.make_async_remote_copy(
                src_ref=p_ref,
                dst_ref=comm_ref.at[my],
                send_sem=send_sems.at[peer],
                recv_sem=recv_sems.at[peer],
                device_id=(peer,),
                device_id_type=pl.DeviceIdType.MESH,
            )
            rdma.start()

    comm_ref[pl.ds(my, 1)] = p_ref[...][None]

    for peer in range(N_DEV):
        @pl.when(my != peer)
        def _():
            recv = pltpu.make_async_remote_copy(
                src_ref=p_ref,
                dst_ref=comm_ref.at[peer],
                send_sem=send_sems.at[peer],
                recv_sem=recv_sems.at[peer],
                device_id=(peer,),
                device_id_type=pl.DeviceIdType.MESH,
            )
            recv.wait_recv()

    o_ref[...] = jnp.sum(comm_ref[...], axis=0)

    for peer in range(N_DEV):
        @pl.when(my != peer)
        def _():
            send = pltpu.make_async_remote_copy(
                src_ref=p_ref,
                dst_ref=comm_ref.at[my],
                send_sem=send_sems.at[peer],
                recv_sem=recv_sems.at[peer],
                device_id=(peer,),
                device_id_type=pl.DeviceIdType.MESH,
            )
            send.wait_send()


def _all_reduce(p, collective_id):
    return pl.pallas_call(
        _ar_body,
        out_shape=jax.ShapeDtypeStruct((B, D), jnp.float32),
        in_specs=[pl.BlockSpec(memory_space=pltpu.VMEM)],
        out_specs=pl.BlockSpec(memory_space=pltpu.VMEM),
        scratch_shapes=[
            pltpu.VMEM((N_DEV, B, D), jnp.float32),
            pltpu.SemaphoreType.DMA((N_DEV,)),
            pltpu.SemaphoreType.DMA((N_DEV,)),
        ],
        compiler_params=pltpu.CompilerParams(collective_id=collective_id),
    )(p)


def kernel(x, Win0, Wout0, Win1, Wout1, Win2, Wout2):
    for k, (win, wout) in enumerate([(Win0, Wout0), (Win1, Wout1), (Win2, Wout2)]):
        p = _mlp_layer(x, win, wout)
        x = _all_reduce(p, collective_id=k)
    return x


# baseline (device time: 171229 ns/iter reference)
import jax
import jax.numpy as jnp
from jax import lax
from jax.experimental import pallas as pl
from jax.experimental.pallas import tpu as pltpu

N_DEV = 8
B = 64
D = 2048
H_SHARD = 4096
BH = 512


def _layer_body(x_ref, win_ref, wout_ref, o_ref):
    j = pl.program_id(0)
    h = jnp.maximum(
        jnp.dot(x_ref[...], win_ref[...], preferred_element_type=jnp.float32),
        0.0,
    )
    contrib = jnp.dot(h, wout_ref[...], preferred_element_type=jnp.float32)

    @pl.when(j == 0)
    def _():
        o_ref[...] = contrib

    @pl.when(j > 0)
    def _():
        o_ref[...] += contrib


def _mlp_layer(x, win, wout):
    return pl.pallas_call(
        _layer_body,
        grid=(H_SHARD // BH,),
        in_specs=[
            pl.BlockSpec((B, D), lambda j: (0, 0)),
            pl.BlockSpec((D, BH), lambda j: (0, j)),
            pl.BlockSpec((BH, D), lambda j: (j, 0)),
        ],
        out_specs=pl.BlockSpec((B, D), lambda j: (0, 0)),
        out_shape=jax.ShapeDtypeStruct((B, D), jnp.float32),
    )(x, win, wout)


def _ar_body(p_ref, o_ref, comm_ref, send_sems, recv_sems):
    my = lax.axis_index("i")

    barrier_sem = pltpu.get_barrier_semaphore()
    for peer in range(N_DEV):
        @pl.when(my != peer)
        def _():
            pl.semaphore_signal(
                barrier_sem, inc=1,
                device_id=(peer,), device_id_type=pl.DeviceIdType.MESH,
            )
    pl.semaphore_wait(barrier_sem, N_DEV - 1)

    for peer in range(N_DEV):
        @pl.when(my != peer)
        def _():
            rdma = pltpu.make_async_remote_copy(
                src_ref=p_ref,
                dst_ref=comm_ref.at[my],
                send_sem=send_sems.at[peer],
                recv_sem=recv_sems.at[my],
                device_id=(peer,),
                device_id_type=pl.DeviceIdType.MESH,
            )
            rdma.start()

    comm_ref[pl.ds(my, 1)] = p_ref[...][None]

    for peer in range(N_DEV):
        @pl.when(my != peer)
        def _():
            recv = pltpu.make_async_remote_copy(
                src_ref=p_ref,
                dst_ref=comm_ref.at[peer],
                send_sem=send_sems.at[peer],
                recv_sem=recv_sems.at[peer],
                device_id=(peer,),
                device_id_type=pl.DeviceIdType.MESH,
            )
            recv.wait_recv()

    o_ref[...] = jnp.sum(comm_ref[...], axis=0)

    for peer in range(N_DEV):
        @pl.when(my != peer)
        def _():
            send = pltpu.make_async_remote_copy(
                src_ref=p_ref,
                dst_ref=comm_ref.at[my],
                send_sem=send_sems.at[peer],
                recv_sem=recv_sems.at[peer],
                device_id=(peer,),
                device_id_type=pl.DeviceIdType.MESH,
            )
            send.wait_send()


def _all_reduce(p, collective_id):
    return pl.pallas_call(
        _ar_body,
        out_shape=jax.ShapeDtypeStruct((B, D), jnp.float32),
        in_specs=[pl.BlockSpec(memory_space=pltpu.VMEM)],
        out_specs=pl.BlockSpec(memory_space=pltpu.VMEM),
        scratch_shapes=[
            pltpu.VMEM((N_DEV, B, D), jnp.float32),
            pltpu.SemaphoreType.DMA((N_DEV,)),
            pltpu.SemaphoreType.DMA((N_DEV,)),
        ],
        compiler_params=pltpu.CompilerParams(collective_id=collective_id),
    )(p)


def kernel(x, Win0, Wout0, Win1, Wout1, Win2, Wout2):
    for k, (win, wout) in enumerate([(Win0, Wout0), (Win1, Wout1), (Win2, Wout2)]):
        p = _mlp_layer(x, win, wout)
        x = _all_reduce(p, collective_id=k)
    return x


# device time: 107382 ns/iter; 1.5946x vs baseline; 1.5946x over previous
import jax
import jax.numpy as jnp
from jax import lax
from jax.experimental import pallas as pl
from jax.experimental.pallas import tpu as pltpu

N_DEV = 8
B = 64
D = 2048
H_SHARD = 4096
BH = 512


def _layer_body(x_ref, win_ref, wout_ref, o_ref):
    j = pl.program_id(0)
    h = jnp.maximum(
        jnp.dot(x_ref[...], win_ref[...], preferred_element_type=jnp.float32),
        0.0,
    )
    contrib = jnp.dot(h, wout_ref[...], preferred_element_type=jnp.float32)

    @pl.when(j == 0)
    def _():
        o_ref[...] = contrib

    @pl.when(j > 0)
    def _():
        o_ref[...] += contrib


def _mlp_layer(x, win, wout):
    return pl.pallas_call(
        _layer_body,
        grid=(H_SHARD // BH,),
        in_specs=[
            pl.BlockSpec((B, D), lambda j: (0, 0)),
            pl.BlockSpec((D, BH), lambda j: (0, j)),
            pl.BlockSpec((BH, D), lambda j: (j, 0)),
        ],
        out_specs=pl.BlockSpec((B, D), lambda j: (0, 0)),
        out_shape=jax.ShapeDtypeStruct((B, D), jnp.float32),
    )(x, win, wout)


DC = D // N_DEV


def _ar_body(p_ref, o_ref, rs_ref, red_ref,
             rs_send, rs_recv, ag_send, ag_recv):
    my = lax.axis_index("i")

    barrier_sem = pltpu.get_barrier_semaphore()
    for peer in range(N_DEV):
        @pl.when(my != peer)
        def _():
            pl.semaphore_signal(
                barrier_sem, inc=1,
                device_id=(peer,), device_id_type=pl.DeviceIdType.MESH,
            )
    pl.semaphore_wait(barrier_sem, N_DEV - 1)

    for peer in range(N_DEV):
        @pl.when(my != peer)
        def _():
            rdma = pltpu.make_async_remote_copy(
                src_ref=p_ref.at[:, pl.ds(peer * DC, DC)],
                dst_ref=rs_ref.at[my],
                send_sem=rs_send.at[peer],
                recv_sem=rs_recv.at[my],
                device_id=(peer,),
                device_id_type=pl.DeviceIdType.MESH,
            )
            rdma.start()

    rs_ref[pl.ds(my, 1)] = p_ref[:, pl.ds(my * DC, DC)][None]

    for peer in range(N_DEV):
        @pl.when(my != peer)
        def _():
            recv = pltpu.make_async_remote_copy(
                src_ref=rs_ref.at[peer],
                dst_ref=rs_ref.at[peer],
                send_sem=rs_send.at[peer],
                recv_sem=rs_recv.at[peer],
                device_id=(peer,),
                device_id_type=pl.DeviceIdType.MESH,
            )
            recv.wait_recv()

    red_ref[...] = jnp.sum(rs_ref[...], axis=0)

    for peer in range(N_DEV):
        @pl.when(my != peer)
        def _():
            rdma = pltpu.make_async_remote_copy(
                src_ref=red_ref,
                dst_ref=o_ref.at[:, pl.ds(my * DC, DC)],
                send_sem=ag_send.at[peer],
                recv_sem=ag_recv.at[my],
                device_id=(peer,),
                device_id_type=pl.DeviceIdType.MESH,
            )
            rdma.start()

    o_ref[:, pl.ds(my * DC, DC)] = red_ref[...]

    for peer in range(N_DEV):
        @pl.when(my != peer)
        def _():
            recv = pltpu.make_async_remote_copy(
                src_ref=red_ref,
                dst_ref=o_ref.at[:, pl.ds(peer * DC, DC)],
                send_sem=ag_send.at[peer],
                recv_sem=ag_recv.at[peer],
                device_id=(peer,),
                device_id_type=pl.DeviceIdType.MESH,
            )
            recv.wait_recv()

    for peer in range(N_DEV):
        @pl.when(my != peer)
        def _():
            s1 = pltpu.make_async_remote_copy(
                src_ref=p_ref.at[:, pl.ds(peer * DC, DC)],
                dst_ref=rs_ref.at[my],
                send_sem=rs_send.at[peer],
                recv_sem=rs_recv.at[my],
                device_id=(peer,),
                device_id_type=pl.DeviceIdType.MESH,
            )
            s1.wait_send()
            s2 = pltpu.make_async_remote_copy(
                src_ref=red_ref,
                dst_ref=o_ref.at[:, pl.ds(my * DC, DC)],
                send_sem=ag_send.at[peer],
                recv_sem=ag_recv.at[my],
                device_id=(peer,),
                device_id_type=pl.DeviceIdType.MESH,
            )
            s2.wait_send()


def _all_reduce(p, collective_id):
    return pl.pallas_call(
        _ar_body,
        out_shape=jax.ShapeDtypeStruct((B, D), jnp.float32),
        in_specs=[pl.BlockSpec(memory_space=pltpu.VMEM)],
        out_specs=pl.BlockSpec(memory_space=pltpu.VMEM),
        scratch_shapes=[
            pltpu.VMEM((N_DEV, B, DC), jnp.float32),
            pltpu.VMEM((B, DC), jnp.float32),
            pltpu.SemaphoreType.DMA((N_DEV,)),
            pltpu.SemaphoreType.DMA((N_DEV,)),
            pltpu.SemaphoreType.DMA((N_DEV,)),
            pltpu.SemaphoreType.DMA((N_DEV,)),
        ],
        compiler_params=pltpu.CompilerParams(collective_id=collective_id),
    )(p)


def kernel(x, Win0, Wout0, Win1, Wout1, Win2, Wout2):
    for k, (win, wout) in enumerate([(Win0, Wout0), (Win1, Wout1), (Win2, Wout2)]):
        p = _mlp_layer(x, win, wout)
        x = _all_reduce(p, collective_id=k)
    return x
